# baseline (speedup 1.0000x reference)
import functools

import jax
import jax.numpy as jnp
from jax import lax
from jax.experimental import pallas as pl
from jax.experimental.pallas import tpu as pltpu
from jax.experimental.pallas import tpu_sc as plsc

_GAMMA = 12.0

_NUM_CORES = 2
_NUM_SUBCORES = 16
_NW = _NUM_CORES * _NUM_SUBCORES
_BATCH = 16384
_D = 64
_BPW = _BATCH // _NW
_CHUNK = 128
_NCHUNK = _BPW // _CHUNK


def _sc_body(hidx_hbm, ridx_hbm, tidx_hbm, ent_hbm, rel_hbm, out_hbm,
             hidx_v, ridx_v, tidx_v, hbuf, rbuf, tbuf, out_v, sem):
    wid = lax.axis_index("s") * _NUM_CORES + lax.axis_index("c")

    pltpu.sync_copy(hidx_hbm.at[wid], hidx_v)
    pltpu.sync_copy(ridx_hbm.at[wid], ridx_v)
    pltpu.sync_copy(tidx_hbm.at[wid], tidx_v)

    copies = []
    for j in range(_NCHUNK):
        dst = pl.ds(j * _CHUNK, _CHUNK)
        copies.append(pltpu.async_copy(ent_hbm.at[hidx_v.at[j]], hbuf.at[dst, :], sem))
        copies.append(pltpu.async_copy(rel_hbm.at[ridx_v.at[j]], rbuf.at[dst, :], sem))
        copies.append(pltpu.async_copy(ent_hbm.at[tidx_v.at[j]], tbuf.at[dst, :], sem))
    for c in copies:
        c.wait()

    last_lane = lax.iota(jnp.int32, 16) == 15

    def row(i, carry):
        acc = jnp.zeros((16,), jnp.float32)
        for c in range(_D // 16):
            sl = pl.ds(c * 16, 16)
            s = hbuf[i, sl] + rbuf[i, sl] - tbuf[i, sl]
            acc = acc + lax.abs(s)
        tot = plsc.cumsum(acc)
        plsc.store_scatter(out_v, [jnp.full((16,), i, jnp.int32)],
                           _GAMMA - tot, mask=last_lane)
        return carry

    lax.fori_loop(0, _BPW, row, 0, unroll=4)

    pltpu.sync_copy(out_v, out_hbm.at[wid])


@jax.jit
def _transe_score(hidx, ridx, tidx, ent_emb, relation_embedding):
    mesh = plsc.VectorSubcoreMesh(core_axis_name="c", subcore_axis_name="s")
    kfn = pl.kernel(
        _sc_body,
        out_type=jax.ShapeDtypeStruct((_NW, _BPW), jnp.float32),
        mesh=mesh,
        compiler_params=pltpu.CompilerParams(
            needs_layout_passes=False, use_tc_tiling_on_sc=False),
        scratch_types=[
            pltpu.VMEM((_NCHUNK, _CHUNK), jnp.int32),
            pltpu.VMEM((_NCHUNK, _CHUNK), jnp.int32),
            pltpu.VMEM((_NCHUNK, _CHUNK), jnp.int32),
            pltpu.VMEM((_BPW, _D), jnp.float32),
            pltpu.VMEM((_BPW, _D), jnp.float32),
            pltpu.VMEM((_BPW, _D), jnp.float32),
            pltpu.VMEM((_BPW,), jnp.float32),
            pltpu.SemaphoreType.DMA,
        ],
    )
    return kfn(hidx, ridx, tidx, ent_emb, relation_embedding)


def kernel(sample, ent_emb, relation_embedding):
    s = sample.astype(jnp.int32)
    hidx = s[:, 0].reshape(_NW, _NCHUNK, _CHUNK)
    ridx = s[:, 1].reshape(_NW, _NCHUNK, _CHUNK)
    tidx = s[:, 2].reshape(_NW, _NCHUNK, _CHUNK)
    out = _transe_score(hidx, ridx, tidx, ent_emb, relation_embedding)
    return out.reshape(_BATCH, 1)
